# baseline (speedup 1.0000x reference)
import functools

import jax
import jax.numpy as jnp
from jax import lax
from jax.experimental import pallas as pl
from jax.experimental.pallas import tpu as pltpu
from jax.experimental.pallas import tpu_sc as plsc

N = 512
S = 32
D = 256
B = N * S
PADN = 520

_info = plsc.get_sparse_core_info()
_NC, _NS = _info.num_cores, _info.num_subcores
NW = _NC * _NS
PW = B // NW
CH = 64
NCHUNK = PW // CH


def _prep_body(x_ref, st_ref, en_ref, c_ref, e1_ref, inv_ref):
    x = x_ref[...]
    st = st_ref[...]
    en = en_ref[...]
    row = lax.broadcasted_iota(jnp.int32, (PADN, N), 0)
    col = lax.broadcasted_iota(jnp.int32, (PADN, N), 1)
    tri = (col < row).astype(jnp.float32)
    c_ref[...] = jnp.dot(tri, x, preferred_element_type=jnp.float32)
    e1_ref[...] = en + 1
    inv_ref[...] = 1.0 / (en - st + 1).astype(jnp.float32)


_prep = pl.pallas_call(
    _prep_body,
    out_shape=[
        jax.ShapeDtypeStruct((PADN, D), jnp.float32),
        jax.ShapeDtypeStruct((N, S), jnp.int32),
        jax.ShapeDtypeStruct((N, S), jnp.float32),
    ],
)


@functools.partial(
    pl.kernel,
    mesh=plsc.VectorSubcoreMesh(core_axis_name="c", subcore_axis_name="s"),
    out_type=jax.ShapeDtypeStruct((B, D), jnp.float32),
    scratch_types=[
        pltpu.VMEM((PW,), jnp.int32),
        pltpu.VMEM((PW,), jnp.int32),
        pltpu.VMEM((PW, 16), jnp.float32),
        pltpu.VMEM((CH, D), jnp.float32),
        pltpu.VMEM((CH, D), jnp.float32),
        pltpu.SemaphoreType.DMA,
        pltpu.SemaphoreType.DMA,
    ],
)
def _sc_pool(c_hbm, s_hbm, e1_hbm, invl_hbm, out_hbm,
             s_v, e1_v, invl_v, a_v, b_v, sem_a, sem_b):
    wid = lax.axis_index("s") * _NC + lax.axis_index("c")
    base = wid * PW
    pltpu.sync_copy(s_hbm.at[pl.ds(base, PW)], s_v)
    pltpu.sync_copy(e1_hbm.at[pl.ds(base, PW)], e1_v)
    pltpu.sync_copy(invl_hbm.at[pl.ds(base, PW)], invl_v)
    for c in range(NCHUNK):
        cp_a = pltpu.async_copy(c_hbm.at[e1_v.at[pl.ds(c * CH, CH)]], a_v, sem_a)
        cp_b = pltpu.async_copy(c_hbm.at[s_v.at[pl.ds(c * CH, CH)]], b_v, sem_b)
        cp_a.wait()
        cp_b.wait()

        def pair(p, carry, c=c):
            iv = invl_v[c * CH + p, :]
            for v in range(D // 16):
                sl = pl.ds(v * 16, 16)
                a_v[p, sl] = (a_v[p, sl] - b_v[p, sl]) * iv + 0.006
            return carry

        lax.fori_loop(0, CH, pair, 0)
        pltpu.sync_copy(a_v, out_hbm.at[pl.ds(base + c * CH, CH)])


def kernel(input, graph):
    starts = graph[..., 0].astype(jnp.int32)
    ends = graph[..., 1].astype(jnp.int32)
    c_tab, e1, inv = _prep(input, starts, ends)
    inv_lanes = jnp.broadcast_to(inv.reshape(B, 1), (B, 16))
    out = _sc_pool(c_tab, starts.reshape(B), e1.reshape(B), inv_lanes)
    return out.reshape(N, S, D)


# speedup vs baseline: 1.1062x; 1.1062x over previous
import functools

import jax
import jax.numpy as jnp
from jax import lax
from jax.experimental import pallas as pl
from jax.experimental.pallas import tpu as pltpu
from jax.experimental.pallas import tpu_sc as plsc

N = 512
S = 32
D = 256
B = N * S
PADN = 520

_info = plsc.get_sparse_core_info()
_NC, _NS = _info.num_cores, _info.num_subcores
NW = _NC * _NS
PW = B // NW
CH = 64
NCHUNK = PW // CH


def _prep_body(x_ref, st_ref, en_ref, c_ref, e1_ref, inv_ref):
    x = x_ref[...]
    st = st_ref[...]
    en = en_ref[...]
    row = lax.broadcasted_iota(jnp.int32, (PADN, N), 0)
    col = lax.broadcasted_iota(jnp.int32, (PADN, N), 1)
    tri = (col < row).astype(jnp.float32)
    c_ref[...] = jnp.dot(tri, x, preferred_element_type=jnp.float32)
    e1_ref[...] = en + 1
    inv_ref[...] = 1.0 / (en - st + 1).astype(jnp.float32)


_prep = pl.pallas_call(
    _prep_body,
    out_shape=[
        jax.ShapeDtypeStruct((PADN, D), jnp.float32),
        jax.ShapeDtypeStruct((N, S), jnp.int32),
        jax.ShapeDtypeStruct((N, S), jnp.float32),
    ],
)


@functools.partial(
    pl.kernel,
    mesh=plsc.VectorSubcoreMesh(core_axis_name="c", subcore_axis_name="s"),
    out_type=jax.ShapeDtypeStruct((B, D), jnp.float32),
    scratch_types=[
        pltpu.VMEM((PW,), jnp.int32),
        pltpu.VMEM((PW,), jnp.int32),
        pltpu.VMEM((2, CH, 16), jnp.float32),
        pltpu.VMEM((2, CH, D), jnp.float32),
        pltpu.VMEM((2, CH, D), jnp.float32),
        pltpu.SemaphoreType.DMA((2,)),
        pltpu.SemaphoreType.DMA((2,)),
        pltpu.SemaphoreType.DMA((2,)),
        pltpu.SemaphoreType.DMA((2,)),
    ],
)
def _sc_pool(c_hbm, s_hbm, e1_hbm, invl_hbm, out_hbm,
             s_v, e1_v, iv_v, a_v, b_v, sem_a, sem_b, sem_i, sem_o):
    wid = lax.axis_index("s") * _NC + lax.axis_index("c")
    base = wid * PW
    pltpu.sync_copy(s_hbm.at[pl.ds(base, PW)], s_v)
    pltpu.sync_copy(e1_hbm.at[pl.ds(base, PW)], e1_v)

    def fire(c):
        buf = c % 2
        return (
            pltpu.async_copy(c_hbm.at[e1_v.at[pl.ds(c * CH, CH)]],
                             a_v.at[buf], sem_a.at[buf]),
            pltpu.async_copy(c_hbm.at[s_v.at[pl.ds(c * CH, CH)]],
                             b_v.at[buf], sem_b.at[buf]),
            pltpu.async_copy(invl_hbm.at[pl.ds(base + c * CH, CH)],
                             iv_v.at[buf], sem_i.at[buf]),
        )

    gathers = {0: fire(0)}
    scatters = {}
    for c in range(NCHUNK):
        buf = c % 2
        if c + 1 < NCHUNK:
            if c >= 1:
                scatters.pop(c - 1).wait()
            gathers[c + 1] = fire(c + 1)
        for cp in gathers.pop(c):
            cp.wait()

        def pair(p, carry, buf=buf):
            iv = iv_v[buf, p, :]
            for v in range(D // 16):
                sl = pl.ds(v * 16, 16)
                a_v[buf, p, sl] = (a_v[buf, p, sl] - b_v[buf, p, sl]) * iv + 0.006
            return carry

        lax.fori_loop(0, CH, pair, 0)
        scatters[c] = pltpu.async_copy(
            a_v.at[buf], out_hbm.at[pl.ds(base + c * CH, CH)], sem_o.at[buf])
    for cp in scatters.values():
        cp.wait()


def kernel(input, graph):
    starts = graph[..., 0].astype(jnp.int32)
    ends = graph[..., 1].astype(jnp.int32)
    c_tab, e1, inv = _prep(input, starts, ends)
    inv_lanes = jnp.broadcast_to(inv.reshape(B, 1), (B, 16))
    out = _sc_pool(c_tab, starts.reshape(B), e1.reshape(B), inv_lanes)
    return out.reshape(N, S, D)
